# baseline (speedup 1.0000x reference)
import functools

import jax
import jax.numpy as jnp
from jax import lax
from jax.experimental import pallas as pl
from jax.experimental.pallas import tpu as pltpu
from jax.experimental.pallas import tpu_sc as plsc

NC = 2
NS = 16
NW = NC * NS

BATCH = 4096
SEQ = 200
EMB = 64
VOCAB = 1000000
TOTAL = BATCH * SEQ
PER_W = TOTAL // NW
CHUNK = 128
CHUNKS = PER_W // CHUNK

PW = 144
PWINS = 217
ROWS_W = PW * PWINS
TAIL = VOCAB - ROWS_W * NW

_MESH = plsc.VectorSubcoreMesh(
    core_axis_name="c", subcore_axis_name="s",
    num_cores=NC, num_subcores=NS)


@functools.partial(
    pl.kernel,
    out_type=jax.ShapeDtypeStruct((VOCAB, 2 * EMB), jnp.float32),
    mesh=_MESH,
    scratch_types=[
        [pltpu.VMEM((PW, EMB), jnp.float32)] * 2,
        [pltpu.VMEM((PW, 2 * EMB), jnp.float32)] * 2,
        [pltpu.SemaphoreType.DMA] * 2,
        [pltpu.SemaphoreType.DMA] * 2,
    ],
    compiler_params=pltpu.CompilerParams(use_tc_tiling_on_sc=True),
)
def _pack(table_hbm, out_hbm, bufs, stages, gsems, ssems):
  wid = lax.axis_index("s") * NC + lax.axis_index("c")
  base = wid * ROWS_W

  def fill(w, b):
    pltpu.async_copy(table_hbm.at[pl.ds(base + w * PW, PW), :],
                     bufs[b], gsems[b])

  def drain_fill(w, b):
    pltpu.make_async_copy(table_hbm.at[pl.ds(base + w * PW, PW), :],
                          bufs[b], gsems[b]).wait()

  def store(w, b):
    pltpu.async_copy(stages[b], out_hbm.at[pl.ds(base + w * PW, PW)],
                     ssems[b])

  def wait_store(w, b):
    pltpu.make_async_copy(stages[b], out_hbm.at[pl.ds(base + w * PW, PW)],
                          ssems[b]).wait()

  def widen(b):
    def row(k, carry):
      for c in range(EMB // 16):
        stages[b][k, pl.ds(c * 16, 16)] = bufs[b][k, pl.ds(c * 16, 16)]
      return carry
    lax.fori_loop(0, PW, row, 0)

  fill(0, 0)

  def outer(w0, carry):
    for b in range(2):
      w = w0 + b
      nb = 1 - b

      @pl.when(w < PWINS)
      def _():
        @pl.when(w + 1 < PWINS)
        def _():
          fill(w + 1, nb)

        drain_fill(w, b)

        @pl.when(w >= 2)
        def _():
          wait_store(w - 2, b)
        widen(b)
        store(w, b)
    return carry

  lax.fori_loop(0, (PWINS + 1) // 2, lambda i, c: outer(i * 2, c), 0)

  wait_store(PWINS - 2, (PWINS - 2) % 2)
  wait_store(PWINS - 1, (PWINS - 1) % 2)

  @pl.when(wid == NW - 1)
  def _():
    tb = ROWS_W * NW
    src = table_hbm.at[pl.ds(tb, TAIL), :]
    pltpu.async_copy(src, bufs[0].at[pl.ds(0, TAIL), :], gsems[0])
    pltpu.make_async_copy(src, bufs[0].at[pl.ds(0, TAIL), :],
                          gsems[0]).wait()

    def row(k, carry):
      for c in range(EMB // 16):
        stages[0][k, pl.ds(c * 16, 16)] = bufs[0][k, pl.ds(c * 16, 16)]
      return carry
    lax.fori_loop(0, TAIL, row, 0)
    dst = out_hbm.at[pl.ds(tb, TAIL)]
    pltpu.async_copy(stages[0].at[pl.ds(0, TAIL)], dst, ssems[0])
    pltpu.make_async_copy(stages[0].at[pl.ds(0, TAIL)], dst,
                          ssems[0]).wait()


@functools.partial(
    pl.kernel,
    out_type=jax.ShapeDtypeStruct((TOTAL, EMB), jnp.float32),
    mesh=_MESH,
    scratch_types=[
        pltpu.VMEM((CHUNKS, CHUNK), jnp.int32),
        [pltpu.VMEM((CHUNK, 2 * EMB), jnp.float32)] * 2,
        [pltpu.VMEM((CHUNK, EMB), jnp.float32)] * 2,
        [pltpu.SemaphoreType.DMA] * 2,
        [pltpu.SemaphoreType.DMA] * 2,
    ],
    compiler_params=pltpu.CompilerParams(use_tc_tiling_on_sc=True),
)
def _gather(word_hbm, table_hbm, out_hbm, idx_v, bufs, stages, gsems, ssems):
  wid = lax.axis_index("s") * NC + lax.axis_index("c")
  pltpu.sync_copy(word_hbm.at[wid], idx_v)

  def fill(j, b):
    pltpu.async_copy(table_hbm.at[idx_v.at[j]], bufs[b], gsems[b])

  def drain_fill(j, b):
    pltpu.make_async_copy(table_hbm.at[idx_v.at[j]], bufs[b],
                          gsems[b]).wait()

  def compact(b):
    def row(k, carry):
      for c in range(EMB // 16):
        stages[b][k, pl.ds(c * 16, 16)] = bufs[b][k, pl.ds(c * 16, 16)]
      return carry
    lax.fori_loop(0, CHUNK, row, 0)

  def store(j, b):
    base = (wid * CHUNKS + j) * CHUNK
    pltpu.async_copy(stages[b], out_hbm.at[pl.ds(base, CHUNK)], ssems[b])

  def wait_store(j, b):
    base = (wid * CHUNKS + j) * CHUNK
    pltpu.make_async_copy(stages[b], out_hbm.at[pl.ds(base, CHUNK)],
                          ssems[b]).wait()

  fill(0, 0)

  def outer(j0, carry):
    for b in range(2):
      j = j0 + b
      nb = 1 - b

      @pl.when(j + 1 < CHUNKS)
      def _():
        fill(j + 1, nb)

      drain_fill(j, b)

      @pl.when(j >= 2)
      def _():
        wait_store(j - 2, b)
      compact(b)
      store(j, b)
    return carry

  lax.fori_loop(0, CHUNKS // 2, lambda i, c: outer(i * 2, c), 0)

  wait_store(CHUNKS - 2, 0)
  wait_store(CHUNKS - 1, 1)


def kernel(WORD, word_table):
  idx = WORD.reshape(NW, CHUNKS, CHUNK)
  table2 = _pack(word_table)
  out = _gather(idx, table2)
  return out.reshape(TOTAL, EMB, 1)


# speedup vs baseline: 1.0025x; 1.0025x over previous
import functools

import jax
import jax.numpy as jnp
from jax import lax
from jax.experimental import pallas as pl
from jax.experimental.pallas import tpu as pltpu
from jax.experimental.pallas import tpu_sc as plsc

NC = 2
NS = 16
NW = NC * NS

BATCH = 4096
SEQ = 200
EMB = 64
TOTAL = BATCH * SEQ
PER_W = TOTAL // NW
CHUNK = 128
CHUNKS = PER_W // CHUNK
PAIRS = CHUNK // 2


def _make_gather():
  mesh = plsc.VectorSubcoreMesh(
      core_axis_name="c", subcore_axis_name="s",
      num_cores=NC, num_subcores=NS)

  @functools.partial(
      pl.kernel,
      out_type=jax.ShapeDtypeStruct((TOTAL, EMB), jnp.float32),
      mesh=mesh,
      scratch_types=[
          pltpu.VMEM((CHUNKS, CHUNK), jnp.int32),
          [pltpu.VMEM((CHUNK,), jnp.int32)] * 2,
          [pltpu.VMEM((CHUNK, 2 * EMB), jnp.float32)] * 2,
          [pltpu.VMEM((CHUNK, EMB), jnp.float32)] * 2,
          [pltpu.SemaphoreType.DMA] * 2,
          [pltpu.SemaphoreType.DMA] * 2,
      ],
      compiler_params=pltpu.CompilerParams(use_tc_tiling_on_sc=True),
  )
  def gather_kernel(word_hbm, table_hbm, out_hbm, idx_v, pidx, bufs, stages,
                    gsems, ssems):
    wid = lax.axis_index("s") * NC + lax.axis_index("c")
    pltpu.sync_copy(word_hbm.at[wid], idx_v)

    def fill(j, b):
      for k in range(CHUNK // 16):
        pidx[b][pl.ds(k * 16, 16)] = idx_v[j, pl.ds(k * 16, 16)] >> 1
      pltpu.async_copy(table_hbm.at[pidx[b]], bufs[b], gsems[b])

    def drain_fill(b):
      pltpu.make_async_copy(table_hbm.at[pidx[b]], bufs[b], gsems[b]).wait()

    def compact(j, b):
      def group(g, carry):
        k0 = g * 16
        srcs = (idx_v[j, pl.ds(k0, 16)] & 1) * EMB
        for l in range(16):
          k = k0 + l
          src = srcs[l]
          for c in range(EMB // 16):
            stages[b][k, pl.ds(c * 16, 16)] = (
                bufs[b][k, pl.ds(src + c * 16, 16)])
        return carry
      lax.fori_loop(0, CHUNK // 16, group, 0)

    def store(j, b):
      base = (wid * CHUNKS + j) * CHUNK
      pltpu.async_copy(stages[b], out_hbm.at[pl.ds(base, CHUNK)], ssems[b])

    def wait_store(j, b):
      base = (wid * CHUNKS + j) * CHUNK
      pltpu.make_async_copy(stages[b], out_hbm.at[pl.ds(base, CHUNK)],
                            ssems[b]).wait()

    fill(0, 0)

    def outer(j0, carry):
      for b in range(2):
        j = j0 + b
        nb = 1 - b

        @pl.when(j + 1 < CHUNKS)
        def _():
          fill(j + 1, nb)

        drain_fill(b)

        @pl.when(j >= 2)
        def _():
          wait_store(j - 2, b)
        compact(j, b)
        store(j, b)
      return carry

    lax.fori_loop(0, CHUNKS // 2, lambda i, c: outer(i * 2, c), 0)

    wait_store(CHUNKS - 2, 0)
    wait_store(CHUNKS - 1, 1)

  return gather_kernel


_gather = _make_gather()


def kernel(WORD, word_table):
  idx = WORD.reshape(NW, CHUNKS, CHUNK)
  table2 = word_table.reshape(word_table.shape[0] // 2, 2 * EMB)
  out = _gather(idx, table2)
  return out.reshape(TOTAL, EMB, 1)
